# Initial kernel scaffold; baseline (speedup 1.0000x reference)
import functools

import jax
import jax.numpy as jnp
from jax import lax
from jax.experimental import pallas as pl
from jax.experimental.pallas import tpu as pltpu
from jax.experimental.pallas import tpu_sc as plsc

B = 16384
DIM = 32
NCH = 8
SIZE = 16777216
LANES = 16
TROWS = SIZE // LANES
RMASK = TROWS - 1
PRIME = (1 << 31) - 1

NC, NS = 2, 16
NW = NC * NS
RPW = B // NW
NB = 16
NBLK = RPW // NB
LPB = NB * NCH


def _fold(s):
    return (s & jnp.uint32(PRIME)) + (s >> 31)


def _rot(n, k):
    low = (n & jnp.uint32((1 << (31 - k)) - 1)) << k
    high = n >> (31 - k)
    return low + high


def _hash(x1, x0, a1, a0, bb):
    s = _fold(_rot(a1 * x1, 26) + a0 * x0)
    s = _fold(s + _rot(a1 * x0, 16))
    s = _fold(s + _rot(a0 * x1, 10))
    s = _fold(s + bb)
    s = _fold(s)
    s = jnp.where(s >= jnp.uint32(PRIME), s - jnp.uint32(PRIME), s)
    return s & jnp.uint32(SIZE - 1)


def _body(x_hbm, tab_hbm, cf_hbm, out_hbm,
          xv, cfv, sidx, widx, o0b, o1b, sstage, wstage, obuf, sem):
    wid = lax.axis_index("s") * NC + lax.axis_index("c")
    base = wid * RPW
    pltpu.sync_copy(x_hbm.at[pl.ds(base, RPW)], xv)
    pltpu.sync_copy(cf_hbm, cfv)
    lanes = lax.iota(jnp.int32, LANES)

    def block(b, carry):
        xu = xv[pl.ds(b * NB, NB)]
        x1 = xu >> 10
        x0 = xu & jnp.uint32(1023)
        for j in range(NCH):
            lkv = lanes * NCH + j
            h0 = _hash(x1, x0, cfv[0, j], cfv[1, j], cfv[2, j])
            r = (h0 >> 4).astype(jnp.int32)
            o0 = (h0 & jnp.uint32(15)).astype(jnp.int32)
            h1 = _hash(x1, x0, cfv[3, j], cfv[4, j], cfv[5, j])
            wr = (h1 >> 4).astype(jnp.int32)
            wo = (h1 & jnp.uint32(15)).astype(jnp.int32)
            zero = lanes * 0
            plsc.store_scatter(sidx, [zero, lkv], r)
            plsc.store_scatter(sidx, [zero + 1, lkv], (r + 1) & RMASK)
            plsc.store_scatter(sidx, [zero + 2, lkv], (r + 2) & RMASK)
            plsc.store_scatter(widx, [lkv], wr)
            plsc.store_scatter(o0b, [lkv], o0)
            plsc.store_scatter(o1b, [lkv], wo)

        c0 = pltpu.async_copy(tab_hbm.at[sidx.at[0]], sstage.at[0], sem)
        c1 = pltpu.async_copy(tab_hbm.at[sidx.at[1]], sstage.at[1], sem)
        c2 = pltpu.async_copy(tab_hbm.at[sidx.at[2]], sstage.at[2], sem)
        c3 = pltpu.async_copy(tab_hbm.at[widx], wstage, sem)
        c0.wait()
        c1.wait()
        c2.wait()
        c3.wait()

        for row in range(NB):
            acc0 = lanes * jnp.float32(0.0)
            acc1 = lanes * jnp.float32(0.0)
            for j in range(NCH):
                lk = row * NCH + j
                o = o0b[lk]
                w = wstage[lk, o1b[lk]]
                p0 = o + lanes
                p1 = p0 + 16
                lkf = lanes * 0 + lk
                g0 = plsc.load_gather(sstage, [p0 >> 4, lkf, p0 & 15])
                g1 = plsc.load_gather(sstage, [p1 >> 4, lkf, p1 & 15])
                acc0 = acc0 + g0 * w
                acc1 = acc1 + g1 * w
            obuf[row, pl.ds(0, LANES)] = acc0 * 2.0
            obuf[row, pl.ds(LANES, LANES)] = acc1 * 2.0
        pltpu.sync_copy(obuf, out_hbm.at[pl.ds(base + b * NB, NB)])
        return carry

    lax.fori_loop(0, NBLK, block, 0)


@jax.jit
def _sc_call(xs, tab2d, cf):
    mesh = plsc.VectorSubcoreMesh(core_axis_name="c", subcore_axis_name="s")
    f = functools.partial(
        pl.kernel,
        out_type=jax.ShapeDtypeStruct((B, DIM), jnp.float32),
        mesh=mesh,
        scratch_types=[
            pltpu.VMEM((RPW,), jnp.uint32),
            pltpu.VMEM((6, LANES), jnp.uint32),
            pltpu.VMEM((3, LPB), jnp.int32),
            pltpu.VMEM((LPB,), jnp.int32),
            pltpu.VMEM((LPB,), jnp.int32),
            pltpu.VMEM((LPB,), jnp.int32),
            pltpu.VMEM((3, LPB, LANES), jnp.float32),
            pltpu.VMEM((LPB, LANES), jnp.float32),
            pltpu.VMEM((NB, DIM), jnp.float32),
            pltpu.SemaphoreType.DMA,
        ],
    )(_body)
    return f(xs, tab2d, cf)


def kernel(x, table0, coeffs0, coeffs1):
    xs = x.astype(jnp.uint32)
    tab2d = table0.reshape(TROWS, LANES)

    def split(c):
        a = c[:, 0]
        return jnp.stack([a >> 16, a & 0xFFFF, c[:, 1]])

    cf = jnp.concatenate([split(coeffs0), split(coeffs1)]).astype(jnp.uint32)
    cf = jnp.pad(cf, ((0, 0), (0, LANES - NCH)))
    return _sc_call(xs, tab2d, cf)


# speedup vs baseline: 143.8924x; 143.8924x over previous
import functools

import jax
import jax.numpy as jnp
from jax import lax
from jax.experimental import pallas as pl
from jax.experimental.pallas import tpu as pltpu
from jax.experimental.pallas import tpu_sc as plsc

B = 16384
DIM = 32
NCH = 8
SIZE = 16777216
LANES = 16
TROWS = SIZE // LANES
RMASK = TROWS - 1
PRIME = (1 << 31) - 1

NC, NS = 2, 16
NW = NC * NS
RPW = B // NW
NB = 16
NBLK = RPW // NB
LPB = NB * NCH


def _fold(s):
    return (s & jnp.uint32(PRIME)) + (s >> 31)


def _rot(n, k):
    low = (n & jnp.uint32((1 << (31 - k)) - 1)) << k
    high = n >> (31 - k)
    return low + high


def _hash(x1, x0, a1, a0, bb):
    s = _fold(_rot(a1 * x1, 26) + a0 * x0)
    s = _fold(s + _rot(a1 * x0, 16))
    s = _fold(s + _rot(a0 * x1, 10))
    s = _fold(s + bb)
    s = _fold(s)
    s = jnp.where(s >= jnp.uint32(PRIME), s - jnp.uint32(PRIME), s)
    return s & jnp.uint32(SIZE - 1)


def _body(x_hbm, tab_hbm, cf_hbm, out_hbm,
          xv, cfv, sidx, widx, o0b, o1b, sstage, wstage, obuf, sem):
    wid = lax.axis_index("s") * jnp.int32(NC) + lax.axis_index("c")
    base = wid * jnp.int32(RPW)
    pltpu.sync_copy(x_hbm.at[pl.ds(base, RPW)], xv)
    pltpu.sync_copy(cf_hbm, cfv)
    lanes = lax.iota(jnp.int32, LANES)

    cfr = [cfv[pl.ds(r * LANES, LANES)] for r in range(6)]
    coef = [[cfr[r][j] for r in range(6)] for j in range(NCH)]

    def block(b, carry):
        xu = xv[pl.ds(b * jnp.int32(NB), NB)]
        x1 = xu >> 10
        x0 = xu & jnp.uint32(1023)
        zero = lanes * 0
        for j in range(NCH):
            a1_0, a0_0, b_0, a1_1, a0_1, b_1 = coef[j]
            h0 = _hash(x1, x0, a1_0, a0_0, b_0)
            r = (h0 >> 4).astype(jnp.int32)
            o0 = (h0 & jnp.uint32(15)).astype(jnp.int32)
            h1 = _hash(x1, x0, a1_1, a0_1, b_1)
            wr = (h1 >> 4).astype(jnp.int32)
            wo = (h1 & jnp.uint32(15)).astype(jnp.int32)
            lkv = lanes * NCH + j
            plsc.store_scatter(sidx, [lkv], r)
            plsc.store_scatter(sidx, [lkv + LPB], (r + 1) & RMASK)
            plsc.store_scatter(sidx, [lkv + 2 * LPB], (r + 2) & RMASK)
            plsc.store_scatter(widx, [lkv], wr)
            plsc.store_scatter(o0b, [lanes * LANES + j], o0)
            plsc.store_scatter(o1b, [lanes * LANES + j], wo)

        i0, i1, i2 = jnp.int32(0), jnp.int32(1), jnp.int32(2)
        c0 = pltpu.async_copy(tab_hbm.at[sidx.at[pl.ds(0, LPB)]], sstage.at[i0], sem)
        c1 = pltpu.async_copy(tab_hbm.at[sidx.at[pl.ds(LPB, LPB)]], sstage.at[i1], sem)
        c2 = pltpu.async_copy(tab_hbm.at[sidx.at[pl.ds(2 * LPB, LPB)]], sstage.at[i2], sem)
        c3 = pltpu.async_copy(tab_hbm.at[widx], wstage, sem)
        c0.wait()
        c1.wait()
        c2.wait()
        c3.wait()

        j8 = lanes & 7
        for row in range(NB):
            ov = o0b[pl.ds(row * LANES, LANES)]
            o1v = o1b[pl.ds(row * LANES, LANES)]
            wv = plsc.load_gather(wstage, [row * NCH + j8, o1v & 15])
            acc0 = lanes * jnp.float32(0.0)
            acc1 = lanes * jnp.float32(0.0)
            for j in range(NCH):
                lk = row * NCH + j
                p0 = ov[j] + lanes
                p1 = p0 + 16
                lkf = zero + lk
                g0 = plsc.load_gather(sstage, [p0 >> 4, lkf, p0 & 15])
                g1 = plsc.load_gather(sstage, [p1 >> 4, lkf, p1 & 15])
                w = wv[j]
                acc0 = acc0 + g0 * w
                acc1 = acc1 + g1 * w
            obuf[row, pl.ds(0, LANES)] = acc0 * 2.0
            obuf[row, pl.ds(LANES, LANES)] = acc1 * 2.0
        pltpu.sync_copy(obuf, out_hbm.at[pl.ds(base + b * jnp.int32(NB), NB)])
        return carry

    lax.fori_loop(jnp.int32(0), jnp.int32(NBLK), block, jnp.int32(0))


@jax.jit
def _sc_call(xs, tab2d, cf):
    mesh = plsc.VectorSubcoreMesh(core_axis_name="c", subcore_axis_name="s")
    f = functools.partial(
        pl.kernel,
        out_type=jax.ShapeDtypeStruct((B, DIM), jnp.float32),
        mesh=mesh,
        scratch_types=[
            pltpu.VMEM((RPW,), jnp.uint32),
            pltpu.VMEM((6 * LANES,), jnp.uint32),
            pltpu.VMEM((3 * LPB,), jnp.int32),
            pltpu.VMEM((LPB,), jnp.int32),
            pltpu.VMEM((NB * LANES,), jnp.int32),
            pltpu.VMEM((NB * LANES,), jnp.int32),
            pltpu.VMEM((3, LPB, LANES), jnp.float32),
            pltpu.VMEM((LPB, LANES), jnp.float32),
            pltpu.VMEM((NB, DIM), jnp.float32),
            pltpu.SemaphoreType.DMA,
        ],
        compiler_params=pltpu.CompilerParams(
            needs_layout_passes=False, use_tc_tiling_on_sc=False),
    )(_body)
    return f(xs, tab2d, cf)


def kernel(x, table0, coeffs0, coeffs1):
    xs = x.astype(jnp.uint32)
    tab2d = table0.reshape(TROWS, LANES)

    def split(c):
        a = c[:, 0]
        return jnp.stack([a >> 16, a & 0xFFFF, c[:, 1]])

    cf = jnp.concatenate([split(coeffs0), split(coeffs1)]).astype(jnp.uint32)
    cf = jnp.pad(cf, ((0, 0), (0, LANES - NCH))).reshape(-1)
    return _sc_call(xs, tab2d, cf)
